# baseline (speedup 1.0000x reference)
import functools

import jax
import jax.numpy as jnp
from jax import lax
from jax.experimental import pallas as pl
from jax.experimental.pallas import tpu as pltpu
from jax.experimental.pallas import tpu_sc as plsc

_S = 64.0
_M = 0.4

_ROWS = 1024
_COLS = 100000
_NC = 2
_NS = 16
_NW = _NC * _NS
_RPW = _ROWS // _NW
_REGION = _RPW * _COLS
_CHUNK = 10000
_NBUF = 4
_NT = _REGION // _CHUNK
_VPC = _CHUNK // 16


def _sc_body(cos_hbm, label_hbm, out_hbm,
             in0, in1, in2, in3, ot0, ot1, ot2, ot3,
             lbl_v, idx_v, val_v,
             is0, is1, is2, is3, os0, os1, os2, os3, gsem):
    in_bufs = (in0, in1, in2, in3)
    out_bufs = (ot0, ot1, ot2, ot3)
    in_sems = (is0, is1, is2, is3)
    out_sems = (os0, os1, os2, os3)

    wid = lax.axis_index("s") * _NC + lax.axis_index("c")
    base = wid * _REGION

    for b in range(_NBUF):
        pltpu.async_copy(cos_hbm.at[pl.ds(base + b * _CHUNK, _CHUNK)],
                         in_bufs[b], in_sems[b])

    def ring_step(g, _):
        for b in range(_NBUF):
            t = g * _NBUF + b
            off = base + t * _CHUNK
            pltpu.make_async_copy(cos_hbm.at[pl.ds(off, _CHUNK)],
                                  in_bufs[b], in_sems[b]).wait()

            @pl.when(g > 0)
            def _():
                pltpu.make_async_copy(out_bufs[b],
                                      out_hbm.at[pl.ds(off, _CHUNK)],
                                      out_sems[b]).wait()

            def scale_step(i, _c):
                sl = pl.ds(i * 16, 16)
                out_bufs[b][sl] = in_bufs[b][sl] * _S
                return 0

            lax.fori_loop(0, _VPC, scale_step, 0, unroll=5)

            pltpu.async_copy(out_bufs[b], out_hbm.at[pl.ds(off, _CHUNK)],
                             out_sems[b])

            @pl.when(t + _NBUF < _NT)
            def _():
                pltpu.async_copy(
                    cos_hbm.at[pl.ds(off + _NBUF * _CHUNK, _CHUNK)],
                    in_bufs[b], in_sems[b])
        return 0

    lax.fori_loop(0, _NT // _NBUF, ring_step, 0)

    for b in range(_NBUF):
        pltpu.make_async_copy(out_bufs[b],
                              out_hbm.at[pl.ds(base, _CHUNK)],
                              out_sems[b]).wait()

    row0 = wid * _RPW
    pltpu.sync_copy(label_hbm.at[pl.ds(row0, _RPW)], lbl_v)
    lanes = lax.iota(jnp.int32, 16)
    for g in range(2):
        sl = pl.ds(g * 16, 16)
        lbl = lbl_v[sl]
        rows = row0 + g * 16 + lanes
        idx_v[sl] = rows * _COLS + jnp.maximum(lbl, 0)
    pltpu.async_copy(out_hbm.at[idx_v], val_v, gsem).wait()
    for g in range(2):
        sl = pl.ds(g * 16, 16)
        margin = jnp.where(lbl_v[sl] >= 0, _S * _M, 0.0)
        val_v[sl] = val_v[sl] - margin
    pltpu.async_copy(val_v, out_hbm.at[idx_v], gsem).wait()


_sc_call = functools.partial(
    pl.kernel,
    out_type=jax.ShapeDtypeStruct((_ROWS * _COLS,), jnp.float32),
    mesh=plsc.VectorSubcoreMesh(core_axis_name="c", subcore_axis_name="s"),
    scratch_types=(
        [pltpu.VMEM((_CHUNK,), jnp.float32) for _ in range(2 * _NBUF)]
        + [pltpu.VMEM((_RPW,), jnp.int32),
           pltpu.VMEM((_RPW,), jnp.int32),
           pltpu.VMEM((_RPW,), jnp.float32)]
        + [pltpu.SemaphoreType.DMA for _ in range(2 * _NBUF + 1)]
    ),
)(_sc_body)


@jax.jit
def kernel(cosine, label):
    rows, n_cols = cosine.shape
    out = _sc_call(cosine.reshape(rows * n_cols), label)
    return out.reshape(rows, n_cols)


# speedup vs baseline: 1.6783x; 1.6783x over previous
import functools

import jax
import jax.numpy as jnp
from jax import lax
from jax.experimental import pallas as pl
from jax.experimental.pallas import tpu as pltpu
from jax.experimental.pallas import tpu_sc as plsc

_S = 64.0
_M = 0.4

_ROWS = 1024
_COLS = 100000
_NC = 2
_NS = 16
_NW = _NC * _NS
_RPW = _ROWS // _NW
_SUB = 8
_NSB = _RPW // _SUB
_W = 2560
_NBUF = 3
_SC_CHUNKS = 39
_SC_COLS = _SC_CHUNKS * _W
_TAIL = _COLS - _SC_COLS


def _sc_body(cos_hbm, label_hbm, out_hbm,
             in0, in1, in2, ot0, ot1, ot2, lbl_v,
             is0, is1, is2, os0, os1, os2):
    in_bufs = (in0, in1, in2)
    out_bufs = (ot0, ot1, ot2)
    in_sems = (is0, is1, is2)
    out_sems = (os0, os1, os2)

    wid = lax.axis_index("s") * _NC + lax.axis_index("c")
    lanes = lax.iota(jnp.int32, 16)
    row_base = wid * _RPW
    pltpu.sync_copy(label_hbm.at[pl.ds(row_base, _RPW)],
                    lbl_v.at[pl.ds(0, _RPW)])

    for sb in range(_NSB):
        row0 = row_base + sb * _SUB
        lbl16 = lbl_v[pl.ds(sb * _SUB, 16)]

        for b in range(_NBUF):
            pltpu.async_copy(
                cos_hbm.at[pl.ds(row0, _SUB), pl.ds(b * _W, _W)],
                in_bufs[b], in_sems[b])

        def ring_step(g, _, row0=row0, lbl16=lbl16):
            for b in range(_NBUF):
                t = g * _NBUF + b
                c0 = t * _W
                pltpu.make_async_copy(
                    cos_hbm.at[pl.ds(row0, _SUB), pl.ds(c0, _W)],
                    in_bufs[b], in_sems[b]).wait()

                @pl.when(g > 0)
                def _():
                    pltpu.make_async_copy(
                        out_bufs[b],
                        out_hbm.at[pl.ds(row0, _SUB), pl.ds(c0, _W)],
                        out_sems[b]).wait()

                for r in range(_SUB):
                    def scale_step(i, _c, b=b, r=r):
                        sl = pl.ds(i * 16, 16)
                        out_bufs[b][r, sl] = in_bufs[b][r, sl] * _S
                        return 0
                    lax.fori_loop(0, _W // 16, scale_step, 0, unroll=8)

                inb = (lbl16 >= c0) & (lbl16 < c0 + _W) & (lanes < _SUB)
                idx_r = jnp.where(inb, lanes, 0)
                idx_c = jnp.where(inb, lbl16 - c0, 0)
                plsc.addupdate_scatter(
                    out_bufs[b], [idx_r, idx_c],
                    jnp.full((16,), -_S * _M, jnp.float32), mask=inb)

                pltpu.async_copy(
                    out_bufs[b],
                    out_hbm.at[pl.ds(row0, _SUB), pl.ds(c0, _W)],
                    out_sems[b])

                @pl.when(t + _NBUF < _SC_CHUNKS)
                def _():
                    pltpu.async_copy(
                        cos_hbm.at[pl.ds(row0, _SUB),
                                   pl.ds(c0 + _NBUF * _W, _W)],
                        in_bufs[b], in_sems[b])
            return 0

        lax.fori_loop(0, _SC_CHUNKS // _NBUF, ring_step, 0)

        for b in range(_NBUF):
            pltpu.make_async_copy(
                out_bufs[b],
                out_hbm.at[pl.ds(row0, _SUB), pl.ds(0, _W)],
                out_sems[b]).wait()


_sc_call = functools.partial(
    pl.kernel,
    out_type=jax.ShapeDtypeStruct((_ROWS, _COLS), jnp.float32),
    mesh=plsc.VectorSubcoreMesh(core_axis_name="c", subcore_axis_name="s"),
    compiler_params=pltpu.CompilerParams(use_tc_tiling_on_sc=True,
                                         needs_layout_passes=False),
    scratch_types=(
        [pltpu.VMEM((_SUB, _W), jnp.float32) for _ in range(2 * _NBUF)]
        + [pltpu.VMEM((2 * _RPW,), jnp.int32)]
        + [pltpu.SemaphoreType.DMA for _ in range(2 * _NBUF)]
    ),
)(_sc_body)


_TAIL_BLK = 256


def _tc_tail_block(cosine_ref, label_ref, out_ref):
    cols = _SC_COLS + jax.lax.broadcasted_iota(
        jnp.int32, (_ROWS, _TAIL_BLK), 1)
    margin = jnp.where(cols == label_ref[...], -_S * _M, 0.0)
    res = cosine_ref[...] * _S + margin.astype(cosine_ref.dtype)
    out_ref[...] = res[:, :_TAIL]


def _tc_tail(cosine, lbl2d):
    return pl.pallas_call(
        _tc_tail_block,
        grid=(1,),
        in_specs=[
            pl.BlockSpec((_ROWS, _TAIL_BLK), lambda i: (0, _SC_COLS // _TAIL_BLK)),
            pl.BlockSpec((_ROWS, 1), lambda i: (0, 0)),
        ],
        out_specs=pl.BlockSpec((_ROWS, _TAIL), lambda i: (0, 0)),
        out_shape=jax.ShapeDtypeStruct((_ROWS, _TAIL), cosine.dtype),
    )(cosine, lbl2d)


@jax.jit
def kernel(cosine, label):
    sc_out = _sc_call(cosine, label)
    tail = _tc_tail(cosine, label.reshape(_ROWS, 1))
    return lax.dynamic_update_slice(sc_out, tail, (0, _SC_COLS))


# speedup vs baseline: 2.8591x; 1.7036x over previous
import functools

import jax
import jax.numpy as jnp
from jax import lax
from jax.experimental import pallas as pl
from jax.experimental.pallas import tpu as pltpu
from jax.experimental.pallas import tpu_sc as plsc

_S = 64.0
_M = 0.4

_ROWS = 1024
_COLS = 100000
_NC = 2
_NS = 16
_NW = _NC * _NS
_RPW = _ROWS // _NW
_SUB = 8
_NSB = _RPW // _SUB
_W = 2560
_NBUF = 3
_SC_CHUNKS = 39
_SC_COLS = _SC_CHUNKS * _W
_TAIL = _COLS - _SC_COLS


def _sc_body(cos_hbm, label_hbm, out_hbm,
             in0, in1, in2, ot0, ot1, ot2, lbl_v,
             is0, is1, is2, os0, os1, os2):
    in_bufs = (in0, in1, in2)
    out_bufs = (ot0, ot1, ot2)
    in_sems = (is0, is1, is2)
    out_sems = (os0, os1, os2)

    wid = lax.axis_index("s") * _NC + lax.axis_index("c")
    lanes = lax.iota(jnp.int32, 16)
    row_base = wid * _RPW
    pltpu.sync_copy(label_hbm.at[pl.ds(row_base, _RPW)],
                    lbl_v.at[pl.ds(0, _RPW)])

    for sb in range(_NSB):
        row0 = row_base + sb * _SUB
        lbl16 = lbl_v[pl.ds(sb * _SUB, 16)]

        for b in range(_NBUF):
            pltpu.async_copy(
                cos_hbm.at[pl.ds(row0, _SUB), pl.ds(b * _W, _W)],
                in_bufs[b], in_sems[b])

        def ring_step(g, _, row0=row0, lbl16=lbl16):
            for b in range(_NBUF):
                t = g * _NBUF + b
                c0 = t * _W
                pltpu.make_async_copy(
                    cos_hbm.at[pl.ds(row0, _SUB), pl.ds(c0, _W)],
                    in_bufs[b], in_sems[b]).wait()

                @pl.when(g > 0)
                def _():
                    pltpu.make_async_copy(
                        out_bufs[b],
                        out_hbm.at[pl.ds(row0, _SUB), pl.ds(c0, _W)],
                        out_sems[b]).wait()

                for r in range(_SUB):
                    @plsc.parallel_loop(0, _W // 16, unroll=8)
                    def _(i, b=b, r=r):
                        sl = pl.ds(i * 16, 16)
                        out_bufs[b][r, sl] = in_bufs[b][r, sl] * _S

                inb = (lbl16 >= c0) & (lbl16 < c0 + _W) & (lanes < _SUB)
                idx_r = jnp.where(inb, lanes, 0)
                idx_c = jnp.where(inb, lbl16 - c0, 0)
                plsc.addupdate_scatter(
                    out_bufs[b], [idx_r, idx_c],
                    jnp.full((16,), -_S * _M, jnp.float32), mask=inb)

                pltpu.async_copy(
                    out_bufs[b],
                    out_hbm.at[pl.ds(row0, _SUB), pl.ds(c0, _W)],
                    out_sems[b])

                @pl.when(t + _NBUF < _SC_CHUNKS)
                def _():
                    pltpu.async_copy(
                        cos_hbm.at[pl.ds(row0, _SUB),
                                   pl.ds(c0 + _NBUF * _W, _W)],
                        in_bufs[b], in_sems[b])
            return 0

        lax.fori_loop(0, _SC_CHUNKS // _NBUF, ring_step, 0)

        for b in range(_NBUF):
            pltpu.make_async_copy(
                out_bufs[b],
                out_hbm.at[pl.ds(row0, _SUB), pl.ds(0, _W)],
                out_sems[b]).wait()


_sc_call = functools.partial(
    pl.kernel,
    out_type=jax.ShapeDtypeStruct((_ROWS, _COLS), jnp.float32),
    mesh=plsc.VectorSubcoreMesh(core_axis_name="c", subcore_axis_name="s"),
    compiler_params=pltpu.CompilerParams(use_tc_tiling_on_sc=True,
                                         needs_layout_passes=False),
    scratch_types=(
        [pltpu.VMEM((_SUB, _W), jnp.float32) for _ in range(2 * _NBUF)]
        + [pltpu.VMEM((2 * _RPW,), jnp.int32)]
        + [pltpu.SemaphoreType.DMA for _ in range(2 * _NBUF)]
    ),
)(_sc_body)


_TAIL_BLK = 256


def _tc_tail_block(cosine_ref, label_ref, out_ref):
    cols = _SC_COLS + jax.lax.broadcasted_iota(
        jnp.int32, (_ROWS, _TAIL_BLK), 1)
    margin = jnp.where(cols == label_ref[...], -_S * _M, 0.0)
    res = cosine_ref[...] * _S + margin.astype(cosine_ref.dtype)
    out_ref[...] = res[:, :_TAIL]


def _tc_tail(cosine, lbl2d):
    return pl.pallas_call(
        _tc_tail_block,
        grid=(1,),
        in_specs=[
            pl.BlockSpec((_ROWS, _TAIL_BLK), lambda i: (0, _SC_COLS // _TAIL_BLK)),
            pl.BlockSpec((_ROWS, 1), lambda i: (0, 0)),
        ],
        out_specs=pl.BlockSpec((_ROWS, _TAIL), lambda i: (0, 0)),
        out_shape=jax.ShapeDtypeStruct((_ROWS, _TAIL), cosine.dtype),
    )(cosine, lbl2d)


@jax.jit
def kernel(cosine, label):
    sc_out = _sc_call(cosine, label)
    tail = _tc_tail(cosine, label.reshape(_ROWS, 1))
    return lax.dynamic_update_slice(sc_out, tail, (0, _SC_COLS))
